# baseline (speedup 1.0000x reference)
import functools

import jax
import jax.numpy as jnp
from jax import lax
from jax.experimental import pallas as pl
from jax.experimental.pallas import tpu as pltpu
from jax.experimental.pallas import tpu_sc as plsc

_NC, _NS = 2, 16
_NW = _NC * _NS
_B = 4096
_D = 64
_BPW = _B // _NW
_EPS = 1e-5


def _sc_gather_body(uid_hbm, iid_hbm, gu_tab, gi_tab, mu_tab, mi_tab,
                    gmf_out, mu_out, mi_out,
                    idx_u, idx_i, gu_v, gi_v, mu_v, mi_v, sem):
    wid = lax.axis_index("s") * _NC + lax.axis_index("c")
    base = wid * _BPW
    pltpu.sync_copy(uid_hbm.at[pl.ds(base, _BPW)], idx_u)
    pltpu.sync_copy(iid_hbm.at[pl.ds(base, _BPW)], idx_i)
    c1 = pltpu.async_copy(gu_tab.at[idx_u], gu_v, sem)
    c2 = pltpu.async_copy(gi_tab.at[idx_i], gi_v, sem)
    c3 = pltpu.async_copy(mu_tab.at[idx_u], mu_v, sem)
    c4 = pltpu.async_copy(mi_tab.at[idx_i], mi_v, sem)
    c3.wait()
    c4.wait()
    pltpu.sync_copy(mu_v, mu_out.at[pl.ds(base, _BPW)])
    pltpu.sync_copy(mi_v, mi_out.at[pl.ds(base, _BPW)])
    c1.wait()
    c2.wait()

    def row(i, carry):
        for j in range(_D // 16):
            sl = pl.ds(j * 16, 16)
            gu_v[i, sl] = gu_v[i, sl] * gi_v[i, sl]
        return carry

    lax.fori_loop(0, _BPW, row, 0)
    pltpu.sync_copy(gu_v, gmf_out.at[pl.ds(base, _BPW)])


@jax.jit
def _sc_gather(user_ids, item_ids, gu_tab, gi_tab, mu_tab, mi_tab):
    mesh = plsc.VectorSubcoreMesh(core_axis_name="c", subcore_axis_name="s")
    f = pl.kernel(
        _sc_gather_body,
        out_type=(
            jax.ShapeDtypeStruct((_B, _D), jnp.float32),
            jax.ShapeDtypeStruct((_B, _D), jnp.float32),
            jax.ShapeDtypeStruct((_B, _D), jnp.float32),
        ),
        mesh=mesh,
        compiler_params=pltpu.CompilerParams(use_tc_tiling_on_sc=False),
        scratch_types=[
            pltpu.VMEM((_BPW,), jnp.int32),
            pltpu.VMEM((_BPW,), jnp.int32),
            pltpu.VMEM((_BPW, _D), jnp.float32),
            pltpu.VMEM((_BPW, _D), jnp.float32),
            pltpu.VMEM((_BPW, _D), jnp.float32),
            pltpu.VMEM((_BPW, _D), jnp.float32),
            pltpu.SemaphoreType.DMA,
        ],
    )
    return f(user_ids, item_ids, gu_tab, gi_tab, mu_tab, mi_tab)


def _mlp_body(gmf_ref, mu_ref, mi_ref,
              w1u_ref, w1i_ref, w2_ref, w3_ref,
              b1_ref, s1_ref, be1_ref, b2_ref, s2_ref, be2_ref,
              b3_ref, s3_ref, be3_ref, wog_ref, woh_ref, bo_ref,
              out_ref):
    mu = mu_ref[...]
    mi = mi_ref[...]
    h = jnp.dot(mu, w1u_ref[...], preferred_element_type=jnp.float32)
    h = h + jnp.dot(mi, w1i_ref[...], preferred_element_type=jnp.float32)
    h = jnp.maximum(h + b1_ref[...], 0.0) * s1_ref[...] + be1_ref[...]
    h = jnp.dot(h, w2_ref[...], preferred_element_type=jnp.float32)
    h = jnp.maximum(h + b2_ref[...], 0.0) * s2_ref[...] + be2_ref[...]
    h = jnp.dot(h, w3_ref[...], preferred_element_type=jnp.float32)
    h = jnp.maximum(h + b3_ref[...], 0.0) * s3_ref[...] + be3_ref[...]
    logit = (jnp.sum(gmf_ref[...] * wog_ref[...], axis=-1)
             + jnp.sum(h * woh_ref[...], axis=-1) + bo_ref[0])
    out_ref[...] = jax.nn.sigmoid(logit)


@jax.jit
def _mlp_tower(gmf, mu, mi, w1u, w1i, w2, w3,
               b1, s1, be1, b2, s2, be2, b3, s3, be3, wog, woh, bo):
    nblk = 4
    rows = _B // nblk
    full = lambda i: (0, 0)
    batch2 = lambda shape: pl.BlockSpec((rows, shape), lambda i: (i, 0))
    return pl.pallas_call(
        _mlp_body,
        grid=(nblk,),
        in_specs=[
            batch2(_D), batch2(_D), batch2(_D),
            pl.BlockSpec((_D, 256), full), pl.BlockSpec((_D, 256), full),
            pl.BlockSpec((256, 128), full), pl.BlockSpec((128, _D), full),
            pl.BlockSpec((1, 256), full), pl.BlockSpec((1, 256), full),
            pl.BlockSpec((1, 256), full),
            pl.BlockSpec((1, 128), full), pl.BlockSpec((1, 128), full),
            pl.BlockSpec((1, 128), full),
            pl.BlockSpec((1, _D), full), pl.BlockSpec((1, _D), full),
            pl.BlockSpec((1, _D), full),
            pl.BlockSpec((1, _D), full), pl.BlockSpec((1, _D), full),
            pl.BlockSpec(memory_space=pltpu.SMEM),
        ],
        out_specs=pl.BlockSpec((rows,), lambda i: (i,)),
        out_shape=jax.ShapeDtypeStruct((_B,), jnp.float32),
    )(gmf, mu, mi, w1u, w1i, w2, w3,
      b1, s1, be1, b2, s2, be2, b3, s3, be3, wog, woh, bo)


def kernel(user_ids, item_ids, gmf_user_tab, gmf_item_tab, mlp_user_tab,
           mlp_item_tab, W1, b1, g1, be1, W2, b2, g2, be2, W3, b3, g3, be3,
           Wo, bo):
    user_ids = user_ids.astype(jnp.int32)
    item_ids = item_ids.astype(jnp.int32)
    gmf, mu, mi = _sc_gather(user_ids, item_ids, gmf_user_tab, gmf_item_tab,
                             mlp_user_tab, mlp_item_tab)
    inv = 1.0 / jnp.sqrt(1.0 + _EPS)
    w1u = W1[:, :_D].T
    w1i = W1[:, _D:].T
    r2 = lambda v: v.reshape(1, -1)
    return _mlp_tower(
        gmf, mu, mi, w1u, w1i, W2.T, W3.T,
        r2(b1), r2(inv * g1), r2(be1),
        r2(b2), r2(inv * g2), r2(be2),
        r2(b3), r2(inv * g3), r2(be3),
        r2(Wo[0, :_D]), r2(Wo[0, _D:]), bo)


# speedup vs baseline: 1.4301x; 1.4301x over previous
import functools

import jax
import jax.numpy as jnp
from jax import lax
from jax.experimental import pallas as pl
from jax.experimental.pallas import tpu as pltpu
from jax.experimental.pallas import tpu_sc as plsc

_NC, _NS = 2, 16
_NW = _NC * _NS
_B = 4096
_D = 64
_BPW = _B // _NW
_EPS = 1e-5
_L = 16


def _sc_gather_body(uid_hbm, iid_hbm, gu_tab, gi_tab, mu_tab, mi_tab,
                    gmf_out, mu_out, mi_out,
                    idx_u, idx_i, gu_v, gi_v, mu_v, mi_v, sem):
    wid = lax.axis_index("s") * _NC + lax.axis_index("c")
    base = wid * _BPW
    pltpu.sync_copy(uid_hbm.at[pl.ds(base, _BPW)], idx_u)
    pltpu.sync_copy(iid_hbm.at[pl.ds(base, _BPW)], idx_i)
    descs = []
    for g in range(_BPW // _L):
        vu = idx_u[pl.ds(g * _L, _L)]
        vi = idx_i[pl.ds(g * _L, _L)]
        for t in range(_L):
            r = g * _L + t
            id_u = vu[t]
            id_i = vi[t]
            descs.append(pltpu.async_copy(gu_tab.at[id_u], gu_v.at[r], sem))
            descs.append(pltpu.async_copy(gi_tab.at[id_i], gi_v.at[r], sem))
            descs.append(pltpu.async_copy(mu_tab.at[id_u], mu_v.at[r], sem))
            descs.append(pltpu.async_copy(mi_tab.at[id_i], mi_v.at[r], sem))
    for d in descs:
        d.wait()
    pltpu.sync_copy(mu_v, mu_out.at[pl.ds(base, _BPW)])
    pltpu.sync_copy(mi_v, mi_out.at[pl.ds(base, _BPW)])

    def row(i, carry):
        for j in range(_D // _L):
            sl = pl.ds(j * _L, _L)
            gu_v[i, sl] = gu_v[i, sl] * gi_v[i, sl]
        return carry

    lax.fori_loop(0, _BPW, row, 0)
    pltpu.sync_copy(gu_v, gmf_out.at[pl.ds(base, _BPW)])


@jax.jit
def _sc_gather(user_ids, item_ids, gu_tab, gi_tab, mu_tab, mi_tab):
    mesh = plsc.VectorSubcoreMesh(core_axis_name="c", subcore_axis_name="s")
    f = pl.kernel(
        _sc_gather_body,
        out_type=(
            jax.ShapeDtypeStruct((_B, _D), jnp.float32),
            jax.ShapeDtypeStruct((_B, _D), jnp.float32),
            jax.ShapeDtypeStruct((_B, _D), jnp.float32),
        ),
        mesh=mesh,
        scratch_types=[
            pltpu.VMEM((_BPW,), jnp.int32),
            pltpu.VMEM((_BPW,), jnp.int32),
            pltpu.VMEM((_BPW, _D), jnp.float32),
            pltpu.VMEM((_BPW, _D), jnp.float32),
            pltpu.VMEM((_BPW, _D), jnp.float32),
            pltpu.VMEM((_BPW, _D), jnp.float32),
            pltpu.SemaphoreType.DMA,
        ],
    )
    return f(user_ids, item_ids, gu_tab, gi_tab, mu_tab, mi_tab)


def _mlp_body(gmf_ref, mu_ref, mi_ref,
              w1u_ref, w1i_ref, w2_ref, w3_ref,
              b1_ref, s1_ref, be1_ref, b2_ref, s2_ref, be2_ref,
              b3_ref, s3_ref, be3_ref, wog_ref, woh_ref, bo_ref,
              out_ref):
    mu = mu_ref[...]
    mi = mi_ref[...]
    h = jnp.dot(mu, w1u_ref[...], preferred_element_type=jnp.float32)
    h = h + jnp.dot(mi, w1i_ref[...], preferred_element_type=jnp.float32)
    h = jnp.maximum(h + b1_ref[...], 0.0) * s1_ref[...] + be1_ref[...]
    h = jnp.dot(h, w2_ref[...], preferred_element_type=jnp.float32)
    h = jnp.maximum(h + b2_ref[...], 0.0) * s2_ref[...] + be2_ref[...]
    h = jnp.dot(h, w3_ref[...], preferred_element_type=jnp.float32)
    h = jnp.maximum(h + b3_ref[...], 0.0) * s3_ref[...] + be3_ref[...]
    logit = (jnp.sum(gmf_ref[...] * wog_ref[...], axis=-1)
             + jnp.sum(h * woh_ref[...], axis=-1) + bo_ref[0])
    out_ref[...] = jax.nn.sigmoid(logit)


@jax.jit
def _mlp_tower(gmf, mu, mi, w1u, w1i, w2, w3,
               b1, s1, be1, b2, s2, be2, b3, s3, be3, wog, woh, bo):
    nblk = 4
    rows = _B // nblk
    full = lambda i: (0, 0)
    batch2 = lambda shape: pl.BlockSpec((rows, shape), lambda i: (i, 0))
    return pl.pallas_call(
        _mlp_body,
        grid=(nblk,),
        in_specs=[
            batch2(_D), batch2(_D), batch2(_D),
            pl.BlockSpec((_D, 256), full), pl.BlockSpec((_D, 256), full),
            pl.BlockSpec((256, 128), full), pl.BlockSpec((128, _D), full),
            pl.BlockSpec((1, 256), full), pl.BlockSpec((1, 256), full),
            pl.BlockSpec((1, 256), full),
            pl.BlockSpec((1, 128), full), pl.BlockSpec((1, 128), full),
            pl.BlockSpec((1, 128), full),
            pl.BlockSpec((1, _D), full), pl.BlockSpec((1, _D), full),
            pl.BlockSpec((1, _D), full),
            pl.BlockSpec((1, _D), full), pl.BlockSpec((1, _D), full),
            pl.BlockSpec(memory_space=pltpu.SMEM),
        ],
        out_specs=pl.BlockSpec((rows,), lambda i: (i,)),
        out_shape=jax.ShapeDtypeStruct((_B,), jnp.float32),
    )(gmf, mu, mi, w1u, w1i, w2, w3,
      b1, s1, be1, b2, s2, be2, b3, s3, be3, wog, woh, bo)


def kernel(user_ids, item_ids, gmf_user_tab, gmf_item_tab, mlp_user_tab,
           mlp_item_tab, W1, b1, g1, be1, W2, b2, g2, be2, W3, b3, g3, be3,
           Wo, bo):
    user_ids = user_ids.astype(jnp.int32)
    item_ids = item_ids.astype(jnp.int32)
    gmf, mu, mi = _sc_gather(user_ids, item_ids, gmf_user_tab, gmf_item_tab,
                             mlp_user_tab, mlp_item_tab)
    inv = 1.0 / jnp.sqrt(1.0 + _EPS)
    w1u = W1[:, :_D].T
    w1i = W1[:, _D:].T
    r2 = lambda v: v.reshape(1, -1)
    return _mlp_tower(
        gmf, mu, mi, w1u, w1i, W2.T, W3.T,
        r2(b1), r2(inv * g1), r2(be1),
        r2(b2), r2(inv * g2), r2(be2),
        r2(b3), r2(inv * g3), r2(be3),
        r2(Wo[0, :_D]), r2(Wo[0, _D:]), bo)
